# Initial kernel scaffold; baseline (speedup 1.0000x reference)
import jax
import jax.numpy as jnp
from jax.experimental import pallas as pl


def kernel(query, key_in, value, Wq, bq, Wk, bk, Wv, bv, Wf, bf):
    raise NotImplementedError("write your pallas kernel here")


# speedup vs baseline: 2.8250x; 2.8250x over previous
import functools

import numpy as np
import jax
import jax.numpy as jnp
from jax import lax
from jax.experimental import pallas as pl
from jax.experimental.pallas import tpu as pltpu
from jax.experimental.pallas import tpu_sc as plsc

B, C, T = 4, 768, 2048
K = 3
FP = 1152
T2 = 4160
R = 128
NROWS = B * C
TT = 512


def _dft_consts():
    t = np.arange(T, dtype=np.float64)[:, None]
    f = np.arange(FP, dtype=np.float64)[None, :]
    ang = 2.0 * np.pi * ((t * f) % T) / T
    valid = (f <= T // 2)
    cosm = np.where(valid, np.cos(ang), 0.0)
    sinm = np.where(valid, np.sin(ang), 0.0)
    w = np.where((f == 0) | (f == T // 2), 1.0, 2.0) * valid
    icos = (w.T * np.cos(ang).T) / (T * T)
    isin = (w.T * np.sin(ang).T) / (T * T)
    return (cosm.astype(np.float32), sinm.astype(np.float32),
            icos.astype(np.float32), isin.astype(np.float32))


_COSM, _SINM, _ICOS, _ISIN = _dft_consts()


def _proj_body(wq, bq, wk, bk, wv, bv, x_q, x_k, x_v, q_out, k_out, v_out):
    q_out[0] = jnp.dot(wq[...], x_q[0], preferred_element_type=jnp.float32) + bq[...]
    k_out[0] = jnp.dot(wk[...], x_k[0], preferred_element_type=jnp.float32) + bk[...]
    v_out[0] = jnp.dot(wv[...], x_v[0], preferred_element_type=jnp.float32) + bv[...]


def _proj(query, key_in, value, Wq, bq, Wk, bk, Wv, bv):
    full = pl.BlockSpec((C, C), lambda b, t: (0, 0))
    bias = pl.BlockSpec((C, 1), lambda b, t: (0, 0))
    xblk = pl.BlockSpec((1, C, TT), lambda b, t: (b, 0, t))
    return pl.pallas_call(
        _proj_body,
        grid=(B, T // TT),
        in_specs=[full, bias, full, bias, full, bias, xblk, xblk, xblk],
        out_specs=[xblk, xblk, xblk],
        out_shape=[jax.ShapeDtypeStruct((B, C, T), jnp.float32)] * 3,
        compiler_params=pltpu.CompilerParams(
            dimension_semantics=("arbitrary", "arbitrary")),
    )(Wq, bq.reshape(C, 1), Wk, bk.reshape(C, 1), Wv, bv.reshape(C, 1),
      query, key_in, value)


def _dup_body(v_in, v2_out):
    v2_out[0, :, 0:T] = v_in[0]
    v2_out[0, :, T:2 * T] = v_in[0]
    v2_out[0, :, 2 * T:T2] = v_in[0, :, 0:T2 - 2 * T]


def _dup(v):
    return pl.pallas_call(
        _dup_body,
        grid=(B,),
        in_specs=[pl.BlockSpec((1, C, T), lambda b: (b, 0, 0))],
        out_specs=pl.BlockSpec((1, C, T2), lambda b: (b, 0, 0)),
        out_shape=jax.ShapeDtypeStruct((B, C, T2), jnp.float32),
    )(v)


def _acfwd_body(q_ref, k_ref, cosm, sinm, pr_out, pi_out):
    qb = q_ref[...]
    kb = k_ref[...]
    hi = jax.lax.Precision.HIGHEST
    fqr = jnp.dot(qb, cosm[...], preferred_element_type=jnp.float32, precision=hi)
    fqs = jnp.dot(qb, sinm[...], preferred_element_type=jnp.float32, precision=hi)
    fkr = jnp.dot(kb, cosm[...], preferred_element_type=jnp.float32, precision=hi)
    fks = jnp.dot(kb, sinm[...], preferred_element_type=jnp.float32, precision=hi)
    pr_out[...] = fqr * fkr + fqs * fks
    pi_out[...] = fqr * fks - fqs * fkr


def _acfwd(q2d, k2d):
    rows = pl.BlockSpec((R, T), lambda r: (r, 0))
    fwd = pl.BlockSpec((T, FP), lambda r: (0, 0))
    pout = pl.BlockSpec((R, FP), lambda r: (r, 0))
    return pl.pallas_call(
        _acfwd_body,
        grid=(NROWS // R,),
        in_specs=[rows, rows, fwd, fwd],
        out_specs=[pout, pout],
        out_shape=[jax.ShapeDtypeStruct((NROWS, FP), jnp.float32)] * 2,
        compiler_params=pltpu.CompilerParams(
            vmem_limit_bytes=100 * 1024 * 1024),
    )(q2d, k2d, jnp.asarray(_COSM), jnp.asarray(_SINM))


def _acinv_body(pr_ref, pi_ref, icos, isin, offs_out, wsel_out):
    hi = jax.lax.Precision.HIGHEST
    ac = (jnp.dot(pr_ref[...], icos[...], preferred_element_type=jnp.float32, precision=hi)
          - jnp.dot(pi_ref[...], isin[...], preferred_element_type=jnp.float32, precision=hi))
    iota = lax.broadcasted_iota(jnp.int32, (R, T), 1)
    acm = ac
    lags, vals = [], []
    for _ in range(K):
        v1 = jnp.max(acm, axis=-1, keepdims=True)
        l1 = jnp.min(jnp.where(acm == v1, iota, T), axis=-1, keepdims=True)
        lags.append(l1)
        vals.append(v1)
        acm = jnp.where(iota == l1, -jnp.inf, acm)
    m = vals[0]
    z = jnp.sum(jnp.exp(ac - m), axis=-1, keepdims=True)
    lag = jnp.concatenate(lags, axis=1)
    val = jnp.concatenate(vals, axis=1)
    offs_out[...] = T - lag
    wsel_out[...] = jnp.exp(val - m) / z


def _acinv(pr, pi):
    pin = pl.BlockSpec((R, FP), lambda r: (r, 0))
    inv = pl.BlockSpec((FP, T), lambda r: (0, 0))
    sel = pl.BlockSpec((R, K), lambda r: (r, 0))
    return pl.pallas_call(
        _acinv_body,
        grid=(NROWS // R,),
        in_specs=[pin, pin, inv, inv],
        out_specs=[sel, sel],
        out_shape=[jax.ShapeDtypeStruct((NROWS, K), jnp.int32),
                   jax.ShapeDtypeStruct((NROWS, K), jnp.float32)],
        compiler_params=pltpu.CompilerParams(
            vmem_limit_bytes=100 * 1024 * 1024),
    )(pr, pi, jnp.asarray(_ICOS), jnp.asarray(_ISIN))


def _acsel(q2d, k2d):
    pr, pi = _acfwd(q2d, k2d)
    return _acinv(pr, pi)


_NC, _NS = 2, 16
_NW = _NC * _NS
_RPW = NROWS // _NW


def _roll_body(v2_hbm, offs_hbm, out_hbm, offs_v, row_v, obuf_v, sem):
    wid = lax.axis_index("s") * _NC + lax.axis_index("c")
    base = wid * _RPW
    pltpu.sync_copy(offs_hbm.at[:, pl.ds(base, _RPW)], offs_v)

    def row_body(j, _):
        rid = base + j
        pltpu.async_copy(v2_hbm.at[rid], row_v, sem).wait()
        for i in range(K):
            chunk = offs_v[i, pl.ds((j >> 4) << 4, 16)]
            lane = j & 15
            sel = jnp.where(lax.iota(jnp.int32, 16) == lane, chunk, 0)
            off = lax.reduce_max(sel, (0,))

            def cp(j2, _):
                obuf_v[pl.ds(j2 * 16, 16)] = row_v[pl.ds(off + j2 * 16, 16)]
                return 0

            lax.fori_loop(0, T // 16, cp, 0, unroll=4)
            pltpu.sync_copy(obuf_v, out_hbm.at[i, rid])
        return 0

    lax.fori_loop(0, _RPW, row_body, 0)


@functools.partial(jax.jit, static_argnums=())
def _roll_sc(v2_rows, offs_t):
    mesh = plsc.VectorSubcoreMesh(core_axis_name="c", subcore_axis_name="s")
    return pl.kernel(
        _roll_body,
        out_type=jax.ShapeDtypeStruct((K, NROWS, T), jnp.float32),
        mesh=mesh,
        compiler_params=pltpu.CompilerParams(use_tc_tiling_on_sc=False,
                                             needs_layout_passes=False),
        scratch_types=[
            pltpu.VMEM((K, _RPW), jnp.int32),
            pltpu.VMEM((T2,), jnp.float32),
            pltpu.VMEM((T,), jnp.float32),
            pltpu.SemaphoreType.DMA,
        ],
    )(v2_rows, offs_t)


def _final_body(wf, bf, wsel, rolled, out):
    ws = wsel[0]
    acc = bf[...]
    for i in range(K):
        wfi = wf[:, i * C:(i + 1) * C] * ws[:, i][None, :]
        acc = acc + jnp.dot(wfi, rolled[i, 0],
                            preferred_element_type=jnp.float32)
    out[0] = acc


def _final(Wf, bf, wsel, rolled):
    return pl.pallas_call(
        _final_body,
        grid=(B, T // TT),
        in_specs=[
            pl.BlockSpec((C, K * C), lambda b, t: (0, 0)),
            pl.BlockSpec((C, 1), lambda b, t: (0, 0)),
            pl.BlockSpec((1, C, K), lambda b, t: (b, 0, 0)),
            pl.BlockSpec((K, 1, C, TT), lambda b, t: (0, b, 0, t)),
        ],
        out_specs=pl.BlockSpec((1, C, TT), lambda b, t: (b, 0, t)),
        out_shape=jax.ShapeDtypeStruct((B, C, T), jnp.float32),
        compiler_params=pltpu.CompilerParams(
            dimension_semantics=("arbitrary", "arbitrary")),
    )(Wf, bf.reshape(C, 1), wsel, rolled)


def kernel(query, key_in, value, Wq, bq, Wk, bk, Wv, bv, Wf, bf):
    q, k, v = _proj(query, key_in, value, Wq, bq, Wk, bk, Wv, bv)
    v2 = _dup(v)
    offs, wsel = _acsel(q.reshape(NROWS, T), k.reshape(NROWS, T))
    rolled = _roll_sc(v2.reshape(NROWS, T2), offs.T)
    return _final(Wf, bf, wsel.reshape(B, C, K),
                  rolled.reshape(K, B, C, T))


# speedup vs baseline: 3.1138x; 1.1022x over previous
import functools

import numpy as np
import jax
import jax.numpy as jnp
from jax import lax
from jax.experimental import pallas as pl
from jax.experimental.pallas import tpu as pltpu
from jax.experimental.pallas import tpu_sc as plsc

B, C, T = 4, 768, 2048
K = 3
FP = 1152
T2 = 4160
R = 128
NROWS = B * C
TT = 512


def _dft_consts():
    t = np.arange(T, dtype=np.float64)[:, None]
    f = np.arange(FP, dtype=np.float64)[None, :]
    ang = 2.0 * np.pi * ((t * f) % T) / T
    valid = (f <= T // 2)
    cosm = np.where(valid, np.cos(ang), 0.0)
    sinm = np.where(valid, np.sin(ang), 0.0)
    w = np.where((f == 0) | (f == T // 2), 1.0, 2.0) * valid
    icos = (w.T * np.cos(ang).T) / (T * T)
    isin = (w.T * np.sin(ang).T) / (T * T)
    return (cosm.astype(np.float32), sinm.astype(np.float32),
            icos.astype(np.float32), isin.astype(np.float32))


_COSM, _SINM, _ICOS, _ISIN = _dft_consts()


def _proj_body(wq, bq, wk, bk, wv, bv, x_q, x_k, x_v, q_out, k_out, v_out):
    q_out[0] = jnp.dot(wq[...], x_q[0], preferred_element_type=jnp.float32) + bq[...]
    k_out[0] = jnp.dot(wk[...], x_k[0], preferred_element_type=jnp.float32) + bk[...]
    v_out[0] = jnp.dot(wv[...], x_v[0], preferred_element_type=jnp.float32) + bv[...]


def _proj(query, key_in, value, Wq, bq, Wk, bk, Wv, bv):
    full = pl.BlockSpec((C, C), lambda b, t: (0, 0))
    bias = pl.BlockSpec((C, 1), lambda b, t: (0, 0))
    xblk = pl.BlockSpec((1, C, TT), lambda b, t: (b, 0, t))
    return pl.pallas_call(
        _proj_body,
        grid=(B, T // TT),
        in_specs=[full, bias, full, bias, full, bias, xblk, xblk, xblk],
        out_specs=[xblk, xblk, xblk],
        out_shape=[jax.ShapeDtypeStruct((B, C, T), jnp.float32)] * 3,
        compiler_params=pltpu.CompilerParams(
            dimension_semantics=("arbitrary", "arbitrary")),
    )(Wq, bq.reshape(C, 1), Wk, bk.reshape(C, 1), Wv, bv.reshape(C, 1),
      query, key_in, value)


def _dup_body(v_in, v2_out):
    v2_out[0, :, 0:T] = v_in[0]
    v2_out[0, :, T:2 * T] = v_in[0]
    v2_out[0, :, 2 * T:T2] = v_in[0, :, 0:T2 - 2 * T]


def _dup(v):
    return pl.pallas_call(
        _dup_body,
        grid=(B,),
        in_specs=[pl.BlockSpec((1, C, T), lambda b: (b, 0, 0))],
        out_specs=pl.BlockSpec((1, C, T2), lambda b: (b, 0, 0)),
        out_shape=jax.ShapeDtypeStruct((B, C, T2), jnp.float32),
    )(v)


def _acfwd_body(q_ref, k_ref, cosm, sinm, pr_out, pi_out):
    qb = q_ref[...]
    kb = k_ref[...]
    hi = jax.lax.Precision.HIGHEST
    fqr = jnp.dot(qb, cosm[...], preferred_element_type=jnp.float32, precision=hi)
    fqs = jnp.dot(qb, sinm[...], preferred_element_type=jnp.float32, precision=hi)
    fkr = jnp.dot(kb, cosm[...], preferred_element_type=jnp.float32, precision=hi)
    fks = jnp.dot(kb, sinm[...], preferred_element_type=jnp.float32, precision=hi)
    pr_out[...] = fqr * fkr + fqs * fks
    pi_out[...] = fqr * fks - fqs * fkr


def _acfwd(q2d, k2d):
    rows = pl.BlockSpec((R, T), lambda r: (r, 0))
    fwd = pl.BlockSpec((T, FP), lambda r: (0, 0))
    pout = pl.BlockSpec((R, FP), lambda r: (r, 0))
    return pl.pallas_call(
        _acfwd_body,
        grid=(NROWS // R,),
        in_specs=[rows, rows, fwd, fwd],
        out_specs=[pout, pout],
        out_shape=[jax.ShapeDtypeStruct((NROWS, FP), jnp.float32)] * 2,
        compiler_params=pltpu.CompilerParams(
            vmem_limit_bytes=100 * 1024 * 1024),
    )(q2d, k2d, jnp.asarray(_COSM), jnp.asarray(_SINM))


def _acinv_body(pr_ref, pi_ref, icos, isin, offs_out, wsel_out):
    hi = jax.lax.Precision.HIGHEST
    ac = (jnp.dot(pr_ref[...], icos[...], preferred_element_type=jnp.float32, precision=hi)
          - jnp.dot(pi_ref[...], isin[...], preferred_element_type=jnp.float32, precision=hi))
    iota = lax.broadcasted_iota(jnp.int32, (R, T), 1)
    acm = ac
    lags, vals = [], []
    for _ in range(K):
        v1 = jnp.max(acm, axis=-1, keepdims=True)
        l1 = jnp.min(jnp.where(acm == v1, iota, T), axis=-1, keepdims=True)
        lags.append(l1)
        vals.append(v1)
        acm = jnp.where(iota == l1, -jnp.inf, acm)
    m = vals[0]
    z = jnp.sum(jnp.exp(ac - m), axis=-1, keepdims=True)
    lag = jnp.concatenate(lags, axis=1)
    val = jnp.concatenate(vals, axis=1)
    offs_out[...] = T - lag
    wsel_out[...] = jnp.exp(val - m) / z


def _acinv(pr, pi):
    pin = pl.BlockSpec((R, FP), lambda r: (r, 0))
    inv = pl.BlockSpec((FP, T), lambda r: (0, 0))
    sel = pl.BlockSpec((R, K), lambda r: (r, 0))
    return pl.pallas_call(
        _acinv_body,
        grid=(NROWS // R,),
        in_specs=[pin, pin, inv, inv],
        out_specs=[sel, sel],
        out_shape=[jax.ShapeDtypeStruct((NROWS, K), jnp.int32),
                   jax.ShapeDtypeStruct((NROWS, K), jnp.float32)],
        compiler_params=pltpu.CompilerParams(
            vmem_limit_bytes=100 * 1024 * 1024),
    )(pr, pi, jnp.asarray(_ICOS), jnp.asarray(_ISIN))


def _acsel(q2d, k2d):
    pr, pi = _acfwd(q2d, k2d)
    return _acinv(pr, pi)


_NC, _NS = 2, 16
_NW = _NC * _NS
_RPW = NROWS // _NW


def _roll_body(v2_hbm, offs_hbm, out_hbm, offs_v, row_v, obuf_v, gsem, ssem):
    wid = lax.axis_index("s") * _NC + lax.axis_index("c")
    base = wid * _RPW
    pltpu.sync_copy(offs_hbm.at[:, pl.ds(base, _RPW)], offs_v)
    pltpu.async_copy(v2_hbm.at[base], row_v.at[0], gsem)

    def row_body(j, _):
        rid = base + j
        cur = j & 1
        @pl.when(j + 1 < _RPW)
        def _():
            pltpu.async_copy(v2_hbm.at[rid + 1], row_v.at[1 - cur], gsem)
        pltpu.make_async_copy(v2_hbm.at[rid], row_v.at[cur], gsem).wait()
        @pl.when(j >= 2)
        def _():
            pltpu.make_async_copy(out_hbm.at[:, 0], obuf_v.at[cur], ssem).wait()
        for i in range(K):
            chunk = offs_v[i, pl.ds((j >> 4) << 4, 16)]
            lane = j & 15
            sel = jnp.where(lax.iota(jnp.int32, 16) == lane, chunk, 0)
            off = lax.reduce_max(sel, (0,))

            def cp(j2, _):
                obuf_v[cur, i, pl.ds(j2 * 16, 16)] = row_v[cur, pl.ds(off + j2 * 16, 16)]
                return 0

            lax.fori_loop(0, T // 16, cp, 0, unroll=8)
            pltpu.async_copy(obuf_v.at[cur, i], out_hbm.at[i, rid], ssem)
        return 0

    lax.fori_loop(0, _RPW, row_body, 0)
    pltpu.make_async_copy(out_hbm.at[:, 0], obuf_v.at[0], ssem).wait()
    pltpu.make_async_copy(out_hbm.at[:, 0], obuf_v.at[1], ssem).wait()


@functools.partial(jax.jit, static_argnums=())
def _roll_sc(v2_rows, offs_t):
    mesh = plsc.VectorSubcoreMesh(core_axis_name="c", subcore_axis_name="s")
    return pl.kernel(
        _roll_body,
        out_type=jax.ShapeDtypeStruct((K, NROWS, T), jnp.float32),
        mesh=mesh,
        compiler_params=pltpu.CompilerParams(use_tc_tiling_on_sc=False,
                                             needs_layout_passes=False),
        scratch_types=[
            pltpu.VMEM((K, _RPW), jnp.int32),
            pltpu.VMEM((2, T2), jnp.float32),
            pltpu.VMEM((2, K, T), jnp.float32),
            pltpu.SemaphoreType.DMA,
            pltpu.SemaphoreType.DMA,
        ],
    )(v2_rows, offs_t)


def _final_body(wf, bf, wsel, rolled, out):
    ws = wsel[0]
    acc = bf[...]
    for i in range(K):
        wfi = wf[:, i * C:(i + 1) * C] * ws[:, i][None, :]
        acc = acc + jnp.dot(wfi, rolled[i, 0],
                            preferred_element_type=jnp.float32)
    out[0] = acc


def _final(Wf, bf, wsel, rolled):
    return pl.pallas_call(
        _final_body,
        grid=(B, T // TT),
        in_specs=[
            pl.BlockSpec((C, K * C), lambda b, t: (0, 0)),
            pl.BlockSpec((C, 1), lambda b, t: (0, 0)),
            pl.BlockSpec((1, C, K), lambda b, t: (b, 0, 0)),
            pl.BlockSpec((K, 1, C, TT), lambda b, t: (0, b, 0, t)),
        ],
        out_specs=pl.BlockSpec((1, C, TT), lambda b, t: (b, 0, t)),
        out_shape=jax.ShapeDtypeStruct((B, C, T), jnp.float32),
        compiler_params=pltpu.CompilerParams(
            dimension_semantics=("arbitrary", "arbitrary")),
    )(Wf, bf.reshape(C, 1), wsel, rolled)


def kernel(query, key_in, value, Wq, bq, Wk, bk, Wv, bv, Wf, bf):
    q, k, v = _proj(query, key_in, value, Wq, bq, Wk, bk, Wv, bv)
    v2 = _dup(v)
    offs, wsel = _acsel(q.reshape(NROWS, T), k.reshape(NROWS, T))
    rolled = _roll_sc(v2.reshape(NROWS, T2), offs.T)
    return _final(Wf, bf, wsel.reshape(B, C, K),
                  rolled.reshape(K, B, C, T))
